# baseline (speedup 1.0000x reference)
import jax
import jax.numpy as jnp
from jax.experimental import pallas as pl


def _quant_body(w1_ref, w0_ref, w1q_ref, w0b_ref):
    w = w1_ref[...]
    bits = jax.lax.bitcast_convert_type(jnp.abs(w), jnp.int32)
    e = (bits >> 23) - 127
    m = bits & 0x7FFFFF
    shift = e + jnp.where(m >= 0x3504F3, 1, 0)
    shift = jnp.clip(shift, -14, 0)
    pow2 = jax.lax.bitcast_convert_type((shift + 127) << 23, jnp.float32)
    w1q = jnp.sign(w) * pow2
    w1q_ref[...] = w1q.astype(jnp.bfloat16)
    w0b_ref[...] = w0_ref[...].astype(jnp.bfloat16)


def _moe_body(x_ref, wg_ref, bg_ref, w0_ref, w1_ref, b0_ref, b1_ref, o_ref):
    x = x_ref[...]
    logits = jax.lax.dot_general(
        x.astype(jnp.bfloat16), wg_ref[...].astype(jnp.bfloat16),
        (((1,), (0,)), ((), ())),
        preferred_element_type=jnp.float32,
    ) + bg_ref[...]
    take1 = logits[:, 1:2] > logits[:, 0:1]
    xb = x.astype(jnp.bfloat16)
    out0 = jax.lax.dot_general(
        xb, w0_ref[...], (((1,), (0,)), ((), ())),
        preferred_element_type=jnp.float32,
    ) + b0_ref[...]
    out1 = jax.lax.dot_general(
        xb, w1_ref[...], (((1,), (0,)), ((), ())),
        preferred_element_type=jnp.float32,
    ) + b1_ref[...]
    o_ref[...] = jnp.where(take1, out1, out0)


@jax.jit
def _run(x, Wg, bg, W0, b0, W1, b1):
    T, C = x.shape
    H = W0.shape[1]
    R = 512

    w1q, w0b = pl.pallas_call(
        _quant_body,
        grid=(4,),
        in_specs=[
            pl.BlockSpec((C, H // 4), lambda j: (0, j)),
            pl.BlockSpec((C, H // 4), lambda j: (0, j)),
        ],
        out_specs=[
            pl.BlockSpec((C, H // 4), lambda j: (0, j)),
            pl.BlockSpec((C, H // 4), lambda j: (0, j)),
        ],
        out_shape=[
            jax.ShapeDtypeStruct((C, H), jnp.bfloat16),
            jax.ShapeDtypeStruct((C, H), jnp.bfloat16),
        ],
    )(W1, W0)

    y = pl.pallas_call(
        _moe_body,
        grid=(T // R,),
        in_specs=[
            pl.BlockSpec((R, C), lambda i: (i, 0)),
            pl.BlockSpec((C, 2), lambda i: (0, 0)),
            pl.BlockSpec((1, 2), lambda i: (0, 0)),
            pl.BlockSpec((C, H), lambda i: (0, 0)),
            pl.BlockSpec((C, H), lambda i: (0, 0)),
            pl.BlockSpec((1, H), lambda i: (0, 0)),
            pl.BlockSpec((1, H), lambda i: (0, 0)),
        ],
        out_specs=pl.BlockSpec((R, H), lambda i: (i, 0)),
        out_shape=jax.ShapeDtypeStruct((T, H), jnp.float32),
    )(x, Wg, bg.reshape(1, 2), w0b, w1q, b0.reshape(1, H), b1.reshape(1, H))
    return y


def kernel(inp, Wg, bg, W0, b0, W1, b1):
    B, N, C = inp.shape
    x = inp.reshape(-1, C)
    y = _run(x, Wg, bg, W0, b0, W1, b1)
    return y.reshape(B, N, -1)
